# baseline (speedup 1.0000x reference)
import functools

import jax
import jax.numpy as jnp
from jax import lax
from jax.experimental import pallas as pl
from jax.experimental.pallas import tpu as pltpu
from jax.experimental.pallas import tpu_sc as plsc

NC = 2
NS = 16
NW = NC * NS
L = 16
B = 128
CB = 16
H = 8
ZR = 224

F32 = jnp.float32
I32 = jnp.int32


def _zero_fill(ref, n):
    z = jnp.zeros((L,), F32)
    for k in range(n // L):
        ref[pl.ds(k * L, L)] = z


def _zero_fill2d(ref, rows):
    z = jnp.zeros((L,), F32)
    for r in range(rows):
        ref[r, :] = z


def _mesh():
    return plsc.VectorSubcoreMesh(
        core_axis_name="c", subcore_axis_name="s", num_cores=NC,
        num_subcores=NS)


def _make_degrees(n_pad, nb0, nb1):
    sl = n_pad // NS

    @functools.partial(
        pl.kernel,
        out_type=jax.ShapeDtypeStruct((NC * 2 * n_pad,), F32),
        mesh=_mesh(),
        compiler_params=pltpu.CompilerParams(use_tc_tiling_on_sc=False),
        scratch_types=[
            pltpu.VMEM((CB, B), I32),
            pltpu.VMEM((CB, B), I32),
            pltpu.VMEM((B,), F32),
            pltpu.VMEM((sl,), F32),
            pltpu.VMEM_SHARED((n_pad,), F32),
            pltpu.VMEM_SHARED((n_pad,), F32),
            pltpu.SemaphoreType.DMA,
        ],
    )
    def deg_kernel(row2d, col2d, out, rbuf, cbuf, ones, zbuf, odeg, ideg,
                   sem):
        c = lax.axis_index("c")
        s = lax.axis_index("s")
        tb = jnp.where(c == 0, s * nb0, NS * nb0 + s * nb1)
        chunks = jnp.where(c == 0, nb0 // CB, nb1 // CB)
        _zero_fill(zbuf, sl)
        one = jnp.ones((L,), F32)
        for k in range(B // L):
            ones[pl.ds(k * L, L)] = one
        pltpu.sync_copy(zbuf, odeg.at[pl.ds(s * sl, sl)])
        pltpu.sync_copy(zbuf, ideg.at[pl.ds(s * sl, sl)])
        plsc.subcore_barrier()

        def chunk(ch, carry):
            base = tb + ch * CB
            pltpu.sync_copy(row2d.at[pl.ds(base, CB)], rbuf)
            pltpu.sync_copy(col2d.at[pl.ds(base, CB)], cbuf)
            sds = []
            for j in range(CB):
                sds.append(pltpu.async_copy(
                    ones, odeg.at[rbuf.at[j]], sem, add=True))
                sds.append(pltpu.async_copy(
                    ones, ideg.at[cbuf.at[j]], sem, add=True))
            for d in sds:
                d.wait()
            return carry

        lax.fori_loop(0, chunks, chunk, 0)
        plsc.subcore_barrier()
        pltpu.sync_copy(odeg.at[pl.ds(s * sl, sl)], zbuf)
        pltpu.sync_copy(zbuf, out.at[pl.ds(c * 2 * n_pad + s * sl, sl)])
        pltpu.sync_copy(ideg.at[pl.ds(s * sl, sl)], zbuf)
        pltpu.sync_copy(
            zbuf, out.at[pl.ds(c * 2 * n_pad + n_pad + s * sl, sl)])

    return deg_kernel


def _make_pass1(n_pad, c_dim, nb0, nb1):
    sl = n_pad // NS
    assert sl % ZR == 0

    @functools.partial(
        pl.kernel,
        out_type=jax.ShapeDtypeStruct((NC, n_pad // ZR, ZR, c_dim), F32),
        mesh=_mesh(),
        compiler_params=pltpu.CompilerParams(use_tc_tiling_on_sc=False),
        scratch_types=[
            pltpu.VMEM((CB, B), I32),
            pltpu.VMEM((CB, B), I32),
            pltpu.VMEM((CB, B, c_dim), F32),
            pltpu.VMEM((ZR, c_dim), F32),
            pltpu.VMEM_SHARED((n_pad, c_dim), F32),
            pltpu.SemaphoreType.DMA,
            pltpu.SemaphoreType.DMA,
        ],
    )
    def pass1_kernel(row2d, col2d, label_a, out, rbuf, cbuf, gbuf, zbuf,
                     acc, gsem, ssem):
        c = lax.axis_index("c")
        s = lax.axis_index("s")
        tb = jnp.where(c == 0, s * nb0, NS * nb0 + s * nb1)
        chunks = jnp.where(c == 0, nb0 // CB, nb1 // CB)
        _zero_fill2d(zbuf, ZR)
        for k in range(sl // ZR):
            pltpu.sync_copy(zbuf, acc.at[pl.ds(s * sl + k * ZR, ZR), :])
        plsc.subcore_barrier()

        def chunk(ch, carry):
            base = tb + ch * CB
            pltpu.sync_copy(row2d.at[pl.ds(base, CB)], rbuf)
            pltpu.sync_copy(col2d.at[pl.ds(base, CB)], cbuf)
            gds = [pltpu.async_copy(label_a.at[rbuf.at[j]], gbuf.at[j], gsem)
                   for j in range(CB)]
            for d in gds:
                d.wait()
            sds = [pltpu.async_copy(gbuf.at[j], acc.at[cbuf.at[j]], ssem,
                                    add=True)
                   for j in range(CB)]
            for d in sds:
                d.wait()
            return carry

        lax.fori_loop(0, chunks, chunk, 0)
        plsc.subcore_barrier()
        for k in range(sl // ZR):
            r0 = s * sl + k * ZR
            pltpu.sync_copy(acc.at[pl.ds(r0, ZR), :], zbuf)
            pltpu.sync_copy(zbuf, out.at[c, r0 // ZR])

    return pass1_kernel


def _make_pass2(n_pad, c_dim, nb0, nb1):
    sl = n_pad // NS
    assert sl % ZR == 0

    @functools.partial(
        pl.kernel,
        out_type=(jax.ShapeDtypeStruct((NC, n_pad // ZR, ZR, c_dim), F32),
                  jax.ShapeDtypeStruct((NC * n_pad,), F32)),
        mesh=_mesh(),
        compiler_params=pltpu.CompilerParams(use_tc_tiling_on_sc=False),
        scratch_types=[
            pltpu.VMEM((CB, B), I32),
            pltpu.VMEM((CB, B), I32),
            pltpu.VMEM((CB, B, c_dim), F32),
            pltpu.VMEM((CB, B), F32),
            pltpu.VMEM((ZR, c_dim), F32),
            pltpu.VMEM((sl,), F32),
            pltpu.VMEM_SHARED((n_pad, c_dim), F32),
            pltpu.VMEM_SHARED((n_pad,), F32),
            pltpu.SemaphoreType.DMA,
            pltpu.SemaphoreType.DMA,
        ],
    )
    def pass2_kernel(row2d, col2d, x1b, b2, t3out, treout, rbuf, cbuf, gbuf,
                     sbuf, zbuf, zs, acc, tre, gsem, ssem):
        c = lax.axis_index("c")
        s = lax.axis_index("s")
        tb = jnp.where(c == 0, s * nb0, NS * nb0 + s * nb1)
        chunks = jnp.where(c == 0, nb0 // CB, nb1 // CB)
        _zero_fill2d(zbuf, ZR)
        for k in range(sl // ZR):
            pltpu.sync_copy(zbuf, acc.at[pl.ds(s * sl + k * ZR, ZR), :])
        _zero_fill(zs, sl)
        pltpu.sync_copy(zs, tre.at[pl.ds(s * sl, sl)])
        plsc.subcore_barrier()

        def chunk(ch, carry):
            base = tb + ch * CB
            pltpu.sync_copy(row2d.at[pl.ds(base, CB)], rbuf)
            pltpu.sync_copy(col2d.at[pl.ds(base, CB)], cbuf)
            gds = []
            for j in range(CB):
                gds.append(pltpu.async_copy(
                    x1b.at[cbuf.at[j]], gbuf.at[j], gsem))
                gds.append(pltpu.async_copy(
                    b2.at[cbuf.at[j]], sbuf.at[j], gsem))
            for d in gds:
                d.wait()
            sds = []
            for j in range(CB):
                sds.append(pltpu.async_copy(
                    gbuf.at[j], acc.at[rbuf.at[j]], ssem, add=True))
                sds.append(pltpu.async_copy(
                    sbuf.at[j], tre.at[rbuf.at[j]], ssem, add=True))
            for d in sds:
                d.wait()
            return carry

        lax.fori_loop(0, chunks, chunk, 0)
        plsc.subcore_barrier()
        for k in range(sl // ZR):
            r0 = s * sl + k * ZR
            pltpu.sync_copy(acc.at[pl.ds(r0, ZR), :], zbuf)
            pltpu.sync_copy(zbuf, t3out.at[c, r0 // ZR])
        pltpu.sync_copy(tre.at[pl.ds(s * sl, sl)], zs)
        pltpu.sync_copy(zs, treout.at[pl.ds(c * n_pad + s * sl, sl)])

    return pass2_kernel


def _tc_scales(degpart, label_p, n_pad, c_dim, blk):
    def body(dp_ref, lab_ref, la_ref, a_ref, b_ref, a2_ref, b2_ref):
        dp = dp_ref[...]
        od = dp[0, 0, :] + dp[1, 0, :] + 1.0
        idg = dp[0, 1, :] + dp[1, 1, :] + 1.0
        a = lax.rsqrt(od)
        b = lax.rsqrt(idg)
        a_ref[...] = a[:, None]
        b_ref[...] = b[:, None]
        a2_ref[...] = (1.0 / od)[:, None]
        b2_ref[...] = (1.0 / idg)[:, None]
        la_ref[...] = a[:, None] * lab_ref[...]

    n_blk = n_pad // blk
    col = jax.ShapeDtypeStruct((n_pad, 1), F32)
    return pl.pallas_call(
        body,
        grid=(n_blk,),
        in_specs=[
            pl.BlockSpec((NC, 2, blk), lambda i: (0, 0, i)),
            pl.BlockSpec((blk, c_dim), lambda i: (i, 0)),
        ],
        out_specs=[
            pl.BlockSpec((blk, c_dim), lambda i: (i, 0)),
            pl.BlockSpec((blk, 1), lambda i: (i, 0)),
            pl.BlockSpec((blk, 1), lambda i: (i, 0)),
            pl.BlockSpec((blk, 1), lambda i: (i, 0)),
            pl.BlockSpec((blk, 1), lambda i: (i, 0)),
        ],
        out_shape=[jax.ShapeDtypeStruct((n_pad, c_dim), F32), col, col, col,
                   col],
    )(degpart, label_p)


def _tc_x1(t1part, b_col, b2_col, n_pad, c_dim, blk):
    def body(tp_ref, b_ref, b2_ref, x1_ref, x1b_ref):
        t1 = tp_ref[0] + tp_ref[1]
        x1_ref[...] = b_ref[...] * t1
        x1b_ref[...] = b2_ref[...] * t1

    n_blk = n_pad // blk
    return pl.pallas_call(
        body,
        grid=(n_blk,),
        in_specs=[
            pl.BlockSpec((NC, blk, c_dim), lambda i: (0, i, 0)),
            pl.BlockSpec((blk, 1), lambda i: (i, 0)),
            pl.BlockSpec((blk, 1), lambda i: (i, 0)),
        ],
        out_specs=[
            pl.BlockSpec((blk, c_dim), lambda i: (i, 0)),
            pl.BlockSpec((blk, c_dim), lambda i: (i, 0)),
        ],
        out_shape=[jax.ShapeDtypeStruct((n_pad, c_dim), F32),
                   jax.ShapeDtypeStruct((n_pad, c_dim), F32)],
    )(t1part, b_col, b2_col)


def _tc_finish(t3part, trepart, a_col, a2_col, x1, label_p, n_pad, c_dim,
               blk):
    def body(tp_ref, trp_ref, a_ref, a2_ref, x1_ref, lab_ref, o_ref):
        t3 = tp_ref[0] + tp_ref[1]
        tre = trp_ref[0] + trp_ref[1]
        x3 = a_ref[...] * t3 - lab_ref[...] * (a2_ref[...] * tre)
        x1 = x1_ref[...]
        o_ref[...] = jnp.concatenate([x3 - x1, x3, x1], axis=1)

    n_blk = n_pad // blk
    return pl.pallas_call(
        body,
        grid=(n_blk,),
        in_specs=[
            pl.BlockSpec((NC, blk, c_dim), lambda i: (0, i, 0)),
            pl.BlockSpec((NC, blk, 1), lambda i: (0, i, 0)),
            pl.BlockSpec((blk, 1), lambda i: (i, 0)),
            pl.BlockSpec((blk, 1), lambda i: (i, 0)),
            pl.BlockSpec((blk, c_dim), lambda i: (i, 0)),
            pl.BlockSpec((blk, c_dim), lambda i: (i, 0)),
        ],
        out_specs=pl.BlockSpec((blk, 3 * c_dim), lambda i: (i, 0)),
        out_shape=jax.ShapeDtypeStruct((n_pad, 3 * c_dim), F32),
    )(t3part, trepart, a_col, a2_col, x1, label_p)


def kernel(x, edge_index, edge_weight, label, is_direct):
    n = label.shape[0]
    c_dim = label.shape[1]
    e = edge_index.shape[1]

    nb_t = -(-e // (NW * B))
    nb_t = -(-nb_t // CB) * CB
    e_pad = NW * nb_t * B
    nb0 = int(0.38 * 2 * nb_t) // CB * CB
    nb1 = 2 * nb_t - nb0

    n_pad = -(-(n + 1) // (NS * ZR)) * (NS * ZR)

    row = edge_index[0]
    col = edge_index[1]
    pad = jnp.full((e_pad - e,), n, I32)
    row2d = jnp.concatenate([row, pad]).reshape(e_pad // B, B)
    col2d = jnp.concatenate([col, pad]).reshape(e_pad // B, B)
    label_p = jnp.zeros((n_pad, c_dim), F32).at[:n].set(label)

    blk = n_pad // 32

    degflat = _make_degrees(n_pad, nb0, nb1)(row2d, col2d)
    degpart = degflat.reshape(NC, 2, n_pad)
    label_a, a_col, b_col, a2_col, b2_col = _tc_scales(
        degpart, label_p, n_pad, c_dim, n_pad // 8)
    t1part = _make_pass1(n_pad, c_dim, nb0, nb1)(row2d, col2d, label_a)
    x1, x1b = _tc_x1(t1part.reshape(NC, n_pad, c_dim), b_col, b2_col,
                     n_pad, c_dim, blk)
    b2_flat = b2_col.reshape(n_pad)
    t3part, treflat = _make_pass2(n_pad, c_dim, nb0, nb1)(
        row2d, col2d, x1b, b2_flat)
    out = _tc_finish(t3part.reshape(NC, n_pad, c_dim),
                     treflat.reshape(NC, n_pad, 1), a_col, a2_col,
                     x1, label_p, n_pad, c_dim, blk)
    return out[:n]


# speedup vs baseline: 1.1371x; 1.1371x over previous
import functools

import jax
import jax.numpy as jnp
from jax import lax
from jax.experimental import pallas as pl
from jax.experimental.pallas import tpu as pltpu
from jax.experimental.pallas import tpu_sc as plsc

NC = 2
NS = 16
NW = NC * NS
L = 16
B = 128
CB = 16
H = 8
ZR = 224

F32 = jnp.float32
I32 = jnp.int32


def _zero_fill(ref, n):
    z = jnp.zeros((L,), F32)
    for k in range(n // L):
        ref[pl.ds(k * L, L)] = z


def _zero_fill2d(ref, rows):
    z = jnp.zeros((L,), F32)
    for r in range(rows):
        ref[r, :] = z


def _mesh():
    return plsc.VectorSubcoreMesh(
        core_axis_name="c", subcore_axis_name="s", num_cores=NC,
        num_subcores=NS)


def _make_degrees(n_pad, nb0, nb1):
    sl = n_pad // NS

    @functools.partial(
        pl.kernel,
        out_type=jax.ShapeDtypeStruct((NC * 2 * n_pad,), F32),
        mesh=_mesh(),
        compiler_params=pltpu.CompilerParams(use_tc_tiling_on_sc=False),
        scratch_types=[
            pltpu.VMEM((CB, B), I32),
            pltpu.VMEM((CB, B), I32),
            pltpu.VMEM((B,), F32),
            pltpu.VMEM((sl,), F32),
            pltpu.VMEM_SHARED((n_pad,), F32),
            pltpu.VMEM_SHARED((n_pad,), F32),
            pltpu.SemaphoreType.DMA,
        ],
    )
    def deg_kernel(row2d, col2d, out, rbuf, cbuf, ones, zbuf, odeg, ideg,
                   sem):
        c = lax.axis_index("c")
        s = lax.axis_index("s")
        tb = jnp.where(c == 0, s * nb0, NS * nb0 + s * nb1)
        chunks = jnp.where(c == 0, nb0 // CB, nb1 // CB)
        _zero_fill(zbuf, sl)
        one = jnp.ones((L,), F32)
        for k in range(B // L):
            ones[pl.ds(k * L, L)] = one
        pltpu.sync_copy(zbuf, odeg.at[pl.ds(s * sl, sl)])
        pltpu.sync_copy(zbuf, ideg.at[pl.ds(s * sl, sl)])
        plsc.subcore_barrier()

        def chunk(ch, carry):
            base = tb + ch * CB
            pltpu.sync_copy(row2d.at[pl.ds(base, CB)], rbuf)
            pltpu.sync_copy(col2d.at[pl.ds(base, CB)], cbuf)
            sds = []
            for j in range(CB):
                sds.append(pltpu.async_copy(
                    ones, odeg.at[rbuf.at[j]], sem, add=True))
                sds.append(pltpu.async_copy(
                    ones, ideg.at[cbuf.at[j]], sem, add=True))
            for d in sds:
                d.wait()
            return carry

        lax.fori_loop(0, chunks, chunk, 0)
        plsc.subcore_barrier()
        pltpu.sync_copy(odeg.at[pl.ds(s * sl, sl)], zbuf)
        pltpu.sync_copy(zbuf, out.at[pl.ds(c * 2 * n_pad + s * sl, sl)])
        pltpu.sync_copy(ideg.at[pl.ds(s * sl, sl)], zbuf)
        pltpu.sync_copy(
            zbuf, out.at[pl.ds(c * 2 * n_pad + n_pad + s * sl, sl)])

    return deg_kernel


def _make_pass1(n_pad, c_dim, nb0, nb1):
    sl = n_pad // NS
    assert sl % ZR == 0

    @functools.partial(
        pl.kernel,
        out_type=jax.ShapeDtypeStruct((NC, n_pad // ZR, ZR, c_dim), F32),
        mesh=_mesh(),
        compiler_params=pltpu.CompilerParams(use_tc_tiling_on_sc=False),
        scratch_types=[
            pltpu.VMEM((CB, B), I32),
            pltpu.VMEM((CB, B), I32),
            pltpu.VMEM((CB, B, c_dim), F32),
            pltpu.VMEM((ZR, c_dim), F32),
            pltpu.VMEM_SHARED((n_pad, c_dim), F32),
            pltpu.SemaphoreType.DMA,
            pltpu.SemaphoreType.DMA,
        ],
    )
    def pass1_kernel(row2d, col2d, label_a, out, rbuf, cbuf, gbuf, zbuf,
                     acc, gsem, ssem):
        c = lax.axis_index("c")
        s = lax.axis_index("s")
        tb = jnp.where(c == 0, s * nb0, NS * nb0 + s * nb1)
        chunks = jnp.where(c == 0, nb0 // CB, nb1 // CB)
        _zero_fill2d(zbuf, ZR)
        for k in range(sl // ZR):
            pltpu.sync_copy(zbuf, acc.at[pl.ds(s * sl + k * ZR, ZR), :])
        plsc.subcore_barrier()

        def chunk(ch, carry):
            base = tb + ch * CB
            pltpu.sync_copy(row2d.at[pl.ds(base, CB)], rbuf)
            pltpu.sync_copy(col2d.at[pl.ds(base, CB)], cbuf)
            gds = [pltpu.async_copy(label_a.at[rbuf.at[j]], gbuf.at[j], gsem)
                   for j in range(CB)]
            for d in gds:
                d.wait()
            sds = [pltpu.async_copy(gbuf.at[j], acc.at[cbuf.at[j]], ssem,
                                    add=True)
                   for j in range(CB)]
            for d in sds:
                d.wait()
            return carry

        lax.fori_loop(0, chunks, chunk, 0)
        plsc.subcore_barrier()
        for k in range(sl // ZR):
            r0 = s * sl + k * ZR
            pltpu.sync_copy(acc.at[pl.ds(r0, ZR), :], zbuf)
            pltpu.sync_copy(zbuf, out.at[c, r0 // ZR])

    return pass1_kernel


def _make_pass2(n_pad, c_dim, nb0, nb1):
    sl = n_pad // NS
    assert sl % ZR == 0

    @functools.partial(
        pl.kernel,
        out_type=(jax.ShapeDtypeStruct((NC, n_pad // ZR, ZR, c_dim), F32),
                  jax.ShapeDtypeStruct((NC * n_pad,), F32)),
        mesh=_mesh(),
        compiler_params=pltpu.CompilerParams(use_tc_tiling_on_sc=False),
        scratch_types=[
            pltpu.VMEM((CB, B), I32),
            pltpu.VMEM((CB, B), I32),
            pltpu.VMEM((CB, B, c_dim), F32),
            pltpu.VMEM((CB, B), F32),
            pltpu.VMEM((ZR, c_dim), F32),
            pltpu.VMEM((sl,), F32),
            pltpu.VMEM_SHARED((n_pad, c_dim), F32),
            pltpu.VMEM_SHARED((n_pad,), F32),
            pltpu.SemaphoreType.DMA,
            pltpu.SemaphoreType.DMA,
        ],
    )
    def pass2_kernel(row2d, col2d, x1b, b2, t3out, treout, rbuf, cbuf, gbuf,
                     sbuf, zbuf, zs, acc, tre, gsem, ssem):
        c = lax.axis_index("c")
        s = lax.axis_index("s")
        tb = jnp.where(c == 0, s * nb0, NS * nb0 + s * nb1)
        chunks = jnp.where(c == 0, nb0 // CB, nb1 // CB)
        _zero_fill2d(zbuf, ZR)
        for k in range(sl // ZR):
            pltpu.sync_copy(zbuf, acc.at[pl.ds(s * sl + k * ZR, ZR), :])
        _zero_fill(zs, sl)
        pltpu.sync_copy(zs, tre.at[pl.ds(s * sl, sl)])
        plsc.subcore_barrier()

        def chunk(ch, carry):
            base = tb + ch * CB
            pltpu.sync_copy(row2d.at[pl.ds(base, CB)], rbuf)
            pltpu.sync_copy(col2d.at[pl.ds(base, CB)], cbuf)
            gds = []
            for j in range(CB):
                gds.append(pltpu.async_copy(
                    x1b.at[cbuf.at[j]], gbuf.at[j], gsem))
                gds.append(pltpu.async_copy(
                    b2.at[cbuf.at[j]], sbuf.at[j], gsem))
            for d in gds:
                d.wait()
            sds = []
            for j in range(CB):
                sds.append(pltpu.async_copy(
                    gbuf.at[j], acc.at[rbuf.at[j]], ssem, add=True))
                sds.append(pltpu.async_copy(
                    sbuf.at[j], tre.at[rbuf.at[j]], ssem, add=True))
            for d in sds:
                d.wait()
            return carry

        lax.fori_loop(0, chunks, chunk, 0)
        plsc.subcore_barrier()
        for k in range(sl // ZR):
            r0 = s * sl + k * ZR
            pltpu.sync_copy(acc.at[pl.ds(r0, ZR), :], zbuf)
            pltpu.sync_copy(zbuf, t3out.at[c, r0 // ZR])
        pltpu.sync_copy(tre.at[pl.ds(s * sl, sl)], zs)
        pltpu.sync_copy(zs, treout.at[pl.ds(c * n_pad + s * sl, sl)])

    return pass2_kernel


def _tc_scales(degpart, label_p, n_pad, c_dim, blk):
    def body(dp_ref, lab_ref, la_ref, a_ref, b_ref, a2_ref, b2_ref):
        dp = dp_ref[...]
        od = dp[0, 0, :] + dp[1, 0, :] + 1.0
        idg = dp[0, 1, :] + dp[1, 1, :] + 1.0
        a = lax.rsqrt(od)
        b = lax.rsqrt(idg)
        a_ref[...] = a[:, None]
        b_ref[...] = b[:, None]
        a2_ref[...] = (1.0 / od)[:, None]
        b2_ref[...] = (1.0 / idg)[:, None]
        la_ref[...] = a[:, None] * lab_ref[...]

    n_blk = n_pad // blk
    col = jax.ShapeDtypeStruct((n_pad, 1), F32)
    return pl.pallas_call(
        body,
        grid=(n_blk,),
        in_specs=[
            pl.BlockSpec((NC, 2, blk), lambda i: (0, 0, i)),
            pl.BlockSpec((blk, c_dim), lambda i: (i, 0)),
        ],
        out_specs=[
            pl.BlockSpec((blk, c_dim), lambda i: (i, 0)),
            pl.BlockSpec((blk, 1), lambda i: (i, 0)),
            pl.BlockSpec((blk, 1), lambda i: (i, 0)),
            pl.BlockSpec((blk, 1), lambda i: (i, 0)),
            pl.BlockSpec((blk, 1), lambda i: (i, 0)),
        ],
        out_shape=[jax.ShapeDtypeStruct((n_pad, c_dim), F32), col, col, col,
                   col],
    )(degpart, label_p)


def _tc_x1(t1part, b_col, b2_col, n_pad, c_dim, blk):
    def body(tp_ref, b_ref, b2_ref, x1_ref, x1b_ref):
        t1 = tp_ref[0] + tp_ref[1]
        x1_ref[...] = b_ref[...] * t1
        x1b_ref[...] = b2_ref[...] * t1

    n_blk = n_pad // blk
    return pl.pallas_call(
        body,
        grid=(n_blk,),
        in_specs=[
            pl.BlockSpec((NC, blk, c_dim), lambda i: (0, i, 0)),
            pl.BlockSpec((blk, 1), lambda i: (i, 0)),
            pl.BlockSpec((blk, 1), lambda i: (i, 0)),
        ],
        out_specs=[
            pl.BlockSpec((blk, c_dim), lambda i: (i, 0)),
            pl.BlockSpec((blk, c_dim), lambda i: (i, 0)),
        ],
        out_shape=[jax.ShapeDtypeStruct((n_pad, c_dim), F32),
                   jax.ShapeDtypeStruct((n_pad, c_dim), F32)],
    )(t1part, b_col, b2_col)


def _tc_finish(t3part, trepart, a_col, a2_col, x1, label_p, n_pad, c_dim,
               blk):
    def body(tp_ref, trp_ref, a_ref, a2_ref, x1_ref, lab_ref, o_ref):
        t3 = tp_ref[0] + tp_ref[1]
        tre = trp_ref[0] + trp_ref[1]
        x3 = a_ref[...] * t3 - lab_ref[...] * (a2_ref[...] * tre)
        x1 = x1_ref[...]
        o_ref[...] = jnp.concatenate([x3 - x1, x3, x1], axis=1)

    n_blk = n_pad // blk
    return pl.pallas_call(
        body,
        grid=(n_blk,),
        in_specs=[
            pl.BlockSpec((NC, blk, c_dim), lambda i: (0, i, 0)),
            pl.BlockSpec((NC, blk, 1), lambda i: (0, i, 0)),
            pl.BlockSpec((blk, 1), lambda i: (i, 0)),
            pl.BlockSpec((blk, 1), lambda i: (i, 0)),
            pl.BlockSpec((blk, c_dim), lambda i: (i, 0)),
            pl.BlockSpec((blk, c_dim), lambda i: (i, 0)),
        ],
        out_specs=pl.BlockSpec((blk, 3 * c_dim), lambda i: (i, 0)),
        out_shape=jax.ShapeDtypeStruct((n_pad, 3 * c_dim), F32),
    )(t3part, trepart, a_col, a2_col, x1, label_p)


def kernel(x, edge_index, edge_weight, label, is_direct):
    n = label.shape[0]
    c_dim = label.shape[1]
    e = edge_index.shape[1]

    nb_t = -(-e // (NW * B))
    nb_t = -(-nb_t // CB) * CB
    e_pad = NW * nb_t * B
    nb0 = int(0.62 * 2 * nb_t) // CB * CB
    nb1 = 2 * nb_t - nb0

    n_pad = -(-(n + 1) // (NS * ZR)) * (NS * ZR)

    row = edge_index[0]
    col = edge_index[1]
    pad = jnp.full((e_pad - e,), n, I32)
    row2d = jnp.concatenate([row, pad]).reshape(e_pad // B, B)
    col2d = jnp.concatenate([col, pad]).reshape(e_pad // B, B)
    label_p = jnp.zeros((n_pad, c_dim), F32).at[:n].set(label)

    blk = n_pad // 32

    degflat = _make_degrees(n_pad, nb0, nb1)(row2d, col2d)
    degpart = degflat.reshape(NC, 2, n_pad)
    label_a, a_col, b_col, a2_col, b2_col = _tc_scales(
        degpart, label_p, n_pad, c_dim, n_pad // 8)
    t1part = _make_pass1(n_pad, c_dim, nb0, nb1)(row2d, col2d, label_a)
    x1, x1b = _tc_x1(t1part.reshape(NC, n_pad, c_dim), b_col, b2_col,
                     n_pad, c_dim, blk)
    b2_flat = b2_col.reshape(n_pad)
    t3part, treflat = _make_pass2(n_pad, c_dim, nb0, nb1)(
        row2d, col2d, x1b, b2_flat)
    out = _tc_finish(t3part.reshape(NC, n_pad, c_dim),
                     treflat.reshape(NC, n_pad, 1), a_col, a2_col,
                     x1, label_p, n_pad, c_dim, blk)
    return out[:n]


# speedup vs baseline: 1.1761x; 1.0343x over previous
import functools

import jax
import jax.numpy as jnp
from jax import lax
from jax.experimental import pallas as pl
from jax.experimental.pallas import tpu as pltpu
from jax.experimental.pallas import tpu_sc as plsc

NC = 2
NS = 16
NW = NC * NS
L = 16
B = 128
CB = 16
H = 8
ZR = 224

F32 = jnp.float32
I32 = jnp.int32


def _zero_fill(ref, n):
    z = jnp.zeros((L,), F32)
    for k in range(n // L):
        ref[pl.ds(k * L, L)] = z


def _zero_fill2d(ref, rows):
    z = jnp.zeros((L,), F32)
    for r in range(rows):
        ref[r, :] = z


def _mesh():
    return plsc.VectorSubcoreMesh(
        core_axis_name="c", subcore_axis_name="s", num_cores=NC,
        num_subcores=NS)


def _make_degrees(n_pad, nb0, nb1):
    sl = n_pad // NS

    @functools.partial(
        pl.kernel,
        out_type=jax.ShapeDtypeStruct((NC * 2 * n_pad,), F32),
        mesh=_mesh(),
        compiler_params=pltpu.CompilerParams(use_tc_tiling_on_sc=False),
        scratch_types=[
            pltpu.VMEM((CB, B), I32),
            pltpu.VMEM((CB, B), I32),
            pltpu.VMEM((B,), F32),
            pltpu.VMEM((sl,), F32),
            pltpu.VMEM_SHARED((n_pad,), F32),
            pltpu.VMEM_SHARED((n_pad,), F32),
            pltpu.SemaphoreType.DMA,
        ],
    )
    def deg_kernel(row2d, col2d, out, rbuf, cbuf, ones, zbuf, odeg, ideg,
                   sem):
        c = lax.axis_index("c")
        s = lax.axis_index("s")
        tb = jnp.where(c == 0, s * nb0, NS * nb0 + s * nb1)
        chunks = jnp.where(c == 0, nb0 // CB, nb1 // CB)
        _zero_fill(zbuf, sl)
        one = jnp.ones((L,), F32)
        for k in range(B // L):
            ones[pl.ds(k * L, L)] = one
        pltpu.sync_copy(zbuf, odeg.at[pl.ds(s * sl, sl)])
        pltpu.sync_copy(zbuf, ideg.at[pl.ds(s * sl, sl)])
        plsc.subcore_barrier()

        def chunk(ch, carry):
            base = tb + ch * CB
            pltpu.sync_copy(row2d.at[pl.ds(base, CB)], rbuf)
            pltpu.sync_copy(col2d.at[pl.ds(base, CB)], cbuf)
            sds = []
            for j in range(CB):
                sds.append(pltpu.async_copy(
                    ones, odeg.at[rbuf.at[j]], sem, add=True))
                sds.append(pltpu.async_copy(
                    ones, ideg.at[cbuf.at[j]], sem, add=True))
            for d in sds:
                d.wait()
            return carry

        lax.fori_loop(0, chunks, chunk, 0)
        plsc.subcore_barrier()
        pltpu.sync_copy(odeg.at[pl.ds(s * sl, sl)], zbuf)
        pltpu.sync_copy(zbuf, out.at[pl.ds(c * 2 * n_pad + s * sl, sl)])
        pltpu.sync_copy(ideg.at[pl.ds(s * sl, sl)], zbuf)
        pltpu.sync_copy(
            zbuf, out.at[pl.ds(c * 2 * n_pad + n_pad + s * sl, sl)])

    return deg_kernel


def _make_pass1(n_pad, c_dim, nb0, nb1):
    sl = n_pad // NS
    assert sl % ZR == 0

    @functools.partial(
        pl.kernel,
        out_type=jax.ShapeDtypeStruct((NC, n_pad // ZR, ZR, c_dim), F32),
        mesh=_mesh(),
        compiler_params=pltpu.CompilerParams(use_tc_tiling_on_sc=False),
        scratch_types=[
            pltpu.VMEM((CB, B), I32),
            pltpu.VMEM((CB, B), I32),
            pltpu.VMEM((CB, B, c_dim), F32),
            pltpu.VMEM((ZR, c_dim), F32),
            pltpu.VMEM_SHARED((n_pad, c_dim), F32),
            pltpu.SemaphoreType.DMA,
            pltpu.SemaphoreType.DMA,
        ],
    )
    def pass1_kernel(row2d, col2d, label_a, out, rbuf, cbuf, gbuf, zbuf,
                     acc, gsem, ssem):
        c = lax.axis_index("c")
        s = lax.axis_index("s")
        tb = jnp.where(c == 0, s * nb0, NS * nb0 + s * nb1)
        chunks = jnp.where(c == 0, nb0 // CB, nb1 // CB)
        _zero_fill2d(zbuf, ZR)
        for k in range(sl // ZR):
            pltpu.sync_copy(zbuf, acc.at[pl.ds(s * sl + k * ZR, ZR), :])
        plsc.subcore_barrier()

        def chunk(ch, carry):
            base = tb + ch * CB
            pltpu.sync_copy(row2d.at[pl.ds(base, CB)], rbuf)
            pltpu.sync_copy(col2d.at[pl.ds(base, CB)], cbuf)
            gds = [pltpu.async_copy(label_a.at[rbuf.at[j]], gbuf.at[j], gsem)
                   for j in range(CB)]
            for d in gds:
                d.wait()
            sds = [pltpu.async_copy(gbuf.at[j], acc.at[cbuf.at[j]], ssem,
                                    add=True)
                   for j in range(CB)]
            for d in sds:
                d.wait()
            return carry

        lax.fori_loop(0, chunks, chunk, 0)
        plsc.subcore_barrier()
        for k in range(sl // ZR):
            r0 = s * sl + k * ZR
            pltpu.sync_copy(acc.at[pl.ds(r0, ZR), :], zbuf)
            pltpu.sync_copy(zbuf, out.at[c, r0 // ZR])

    return pass1_kernel


def _make_pass2(n_pad, c_dim, nb0, nb1):
    sl = n_pad // NS
    assert sl % ZR == 0

    @functools.partial(
        pl.kernel,
        out_type=(jax.ShapeDtypeStruct((NC, n_pad // ZR, ZR, c_dim), F32),
                  jax.ShapeDtypeStruct((NC * n_pad,), F32)),
        mesh=_mesh(),
        compiler_params=pltpu.CompilerParams(use_tc_tiling_on_sc=False),
        scratch_types=[
            pltpu.VMEM((CB, B), I32),
            pltpu.VMEM((CB, B), I32),
            pltpu.VMEM((CB, B, c_dim), F32),
            pltpu.VMEM((CB, B), F32),
            pltpu.VMEM((ZR, c_dim), F32),
            pltpu.VMEM((sl,), F32),
            pltpu.VMEM_SHARED((n_pad, c_dim), F32),
            pltpu.VMEM_SHARED((n_pad,), F32),
            pltpu.SemaphoreType.DMA,
            pltpu.SemaphoreType.DMA,
        ],
    )
    def pass2_kernel(row2d, col2d, x1b, b2, t3out, treout, rbuf, cbuf, gbuf,
                     sbuf, zbuf, zs, acc, tre, gsem, ssem):
        c = lax.axis_index("c")
        s = lax.axis_index("s")
        tb = jnp.where(c == 0, s * nb0, NS * nb0 + s * nb1)
        chunks = jnp.where(c == 0, nb0 // CB, nb1 // CB)
        _zero_fill2d(zbuf, ZR)
        for k in range(sl // ZR):
            pltpu.sync_copy(zbuf, acc.at[pl.ds(s * sl + k * ZR, ZR), :])
        _zero_fill(zs, sl)
        pltpu.sync_copy(zs, tre.at[pl.ds(s * sl, sl)])
        plsc.subcore_barrier()

        def chunk(ch, carry):
            base = tb + ch * CB
            pltpu.sync_copy(row2d.at[pl.ds(base, CB)], rbuf)
            pltpu.sync_copy(col2d.at[pl.ds(base, CB)], cbuf)
            gds = []
            for j in range(CB):
                gds.append(pltpu.async_copy(
                    x1b.at[cbuf.at[j]], gbuf.at[j], gsem))
                gds.append(pltpu.async_copy(
                    b2.at[cbuf.at[j]], sbuf.at[j], gsem))
            for d in gds:
                d.wait()
            sds = []
            for j in range(CB):
                sds.append(pltpu.async_copy(
                    gbuf.at[j], acc.at[rbuf.at[j]], ssem, add=True))
                sds.append(pltpu.async_copy(
                    sbuf.at[j], tre.at[rbuf.at[j]], ssem, add=True))
            for d in sds:
                d.wait()
            return carry

        lax.fori_loop(0, chunks, chunk, 0)
        plsc.subcore_barrier()
        for k in range(sl // ZR):
            r0 = s * sl + k * ZR
            pltpu.sync_copy(acc.at[pl.ds(r0, ZR), :], zbuf)
            pltpu.sync_copy(zbuf, t3out.at[c, r0 // ZR])
        pltpu.sync_copy(tre.at[pl.ds(s * sl, sl)], zs)
        pltpu.sync_copy(zs, treout.at[pl.ds(c * n_pad + s * sl, sl)])

    return pass2_kernel


def _tc_scales(degpart, label_p, n_pad, c_dim, blk):
    def body(dp_ref, lab_ref, la_ref, a_ref, b_ref, a2_ref, b2_ref):
        dp = dp_ref[...]
        od = dp[0, 0, :] + dp[1, 0, :] + 1.0
        idg = dp[0, 1, :] + dp[1, 1, :] + 1.0
        a = lax.rsqrt(od)
        b = lax.rsqrt(idg)
        a_ref[...] = a[:, None]
        b_ref[...] = b[:, None]
        a2_ref[...] = (1.0 / od)[:, None]
        b2_ref[...] = (1.0 / idg)[:, None]
        la_ref[...] = a[:, None] * lab_ref[...]

    n_blk = n_pad // blk
    col = jax.ShapeDtypeStruct((n_pad, 1), F32)
    return pl.pallas_call(
        body,
        grid=(n_blk,),
        in_specs=[
            pl.BlockSpec((NC, 2, blk), lambda i: (0, 0, i)),
            pl.BlockSpec((blk, c_dim), lambda i: (i, 0)),
        ],
        out_specs=[
            pl.BlockSpec((blk, c_dim), lambda i: (i, 0)),
            pl.BlockSpec((blk, 1), lambda i: (i, 0)),
            pl.BlockSpec((blk, 1), lambda i: (i, 0)),
            pl.BlockSpec((blk, 1), lambda i: (i, 0)),
            pl.BlockSpec((blk, 1), lambda i: (i, 0)),
        ],
        out_shape=[jax.ShapeDtypeStruct((n_pad, c_dim), F32), col, col, col,
                   col],
    )(degpart, label_p)


def _tc_x1(t1part, b_col, b2_col, n_pad, c_dim, blk):
    def body(tp_ref, b_ref, b2_ref, x1_ref, x1b_ref):
        t1 = tp_ref[0] + tp_ref[1]
        x1_ref[...] = b_ref[...] * t1
        x1b_ref[...] = b2_ref[...] * t1

    n_blk = n_pad // blk
    return pl.pallas_call(
        body,
        grid=(n_blk,),
        in_specs=[
            pl.BlockSpec((NC, blk, c_dim), lambda i: (0, i, 0)),
            pl.BlockSpec((blk, 1), lambda i: (i, 0)),
            pl.BlockSpec((blk, 1), lambda i: (i, 0)),
        ],
        out_specs=[
            pl.BlockSpec((blk, c_dim), lambda i: (i, 0)),
            pl.BlockSpec((blk, c_dim), lambda i: (i, 0)),
        ],
        out_shape=[jax.ShapeDtypeStruct((n_pad, c_dim), F32),
                   jax.ShapeDtypeStruct((n_pad, c_dim), F32)],
    )(t1part, b_col, b2_col)


def _tc_finish(t3part, trepart, a_col, a2_col, x1, label_p, n_pad, c_dim,
               blk):
    def body(tp_ref, trp_ref, a_ref, a2_ref, x1_ref, lab_ref, o_ref):
        t3 = tp_ref[0] + tp_ref[1]
        tre = trp_ref[0] + trp_ref[1]
        x3 = a_ref[...] * t3 - lab_ref[...] * (a2_ref[...] * tre)
        x1 = x1_ref[...]
        o_ref[...] = jnp.concatenate([x3 - x1, x3, x1], axis=1)

    n_blk = n_pad // blk
    return pl.pallas_call(
        body,
        grid=(n_blk,),
        in_specs=[
            pl.BlockSpec((NC, blk, c_dim), lambda i: (0, i, 0)),
            pl.BlockSpec((NC, blk, 1), lambda i: (0, i, 0)),
            pl.BlockSpec((blk, 1), lambda i: (i, 0)),
            pl.BlockSpec((blk, 1), lambda i: (i, 0)),
            pl.BlockSpec((blk, c_dim), lambda i: (i, 0)),
            pl.BlockSpec((blk, c_dim), lambda i: (i, 0)),
        ],
        out_specs=pl.BlockSpec((blk, 3 * c_dim), lambda i: (i, 0)),
        out_shape=jax.ShapeDtypeStruct((n_pad, 3 * c_dim), F32),
    )(t3part, trepart, a_col, a2_col, x1, label_p)


def kernel(x, edge_index, edge_weight, label, is_direct):
    n = label.shape[0]
    c_dim = label.shape[1]
    e = edge_index.shape[1]

    nb_t = -(-e // (NW * B))
    nb_t = -(-nb_t // CB) * CB
    e_pad = NW * nb_t * B
    nb0 = int(0.66 * 2 * nb_t) // CB * CB
    nb1 = 2 * nb_t - nb0

    n_pad = -(-(n + 1) // (NS * ZR)) * (NS * ZR)

    row = edge_index[0]
    col = edge_index[1]
    pad = jnp.full((e_pad - e,), n, I32)
    row2d = jnp.concatenate([row, pad]).reshape(e_pad // B, B)
    col2d = jnp.concatenate([col, pad]).reshape(e_pad // B, B)
    label_p = jnp.zeros((n_pad, c_dim), F32).at[:n].set(label)

    blk = n_pad // 32

    degflat = _make_degrees(n_pad, nb0, nb1)(row2d, col2d)
    degpart = degflat.reshape(NC, 2, n_pad)
    label_a, a_col, b_col, a2_col, b2_col = _tc_scales(
        degpart, label_p, n_pad, c_dim, n_pad // 8)
    t1part = _make_pass1(n_pad, c_dim, nb0, nb1)(row2d, col2d, label_a)
    x1, x1b = _tc_x1(t1part.reshape(NC, n_pad, c_dim), b_col, b2_col,
                     n_pad, c_dim, blk)
    b2_flat = b2_col.reshape(n_pad)
    t3part, treflat = _make_pass2(n_pad, c_dim, nb0, nb1)(
        row2d, col2d, x1b, b2_flat)
    out = _tc_finish(t3part.reshape(NC, n_pad, c_dim),
                     treflat.reshape(NC, n_pad, 1), a_col, a2_col,
                     x1, label_p, n_pad, c_dim, blk)
    return out[:n]


# speedup vs baseline: 1.2089x; 1.0279x over previous
import functools

import jax
import jax.numpy as jnp
from jax import lax
from jax.experimental import pallas as pl
from jax.experimental.pallas import tpu as pltpu
from jax.experimental.pallas import tpu_sc as plsc

NC = 2
NS = 16
NW = NC * NS
L = 16
B = 128
CB = 16
H = 8
ZR = 224

F32 = jnp.float32
I32 = jnp.int32


def _zero_fill(ref, n):
    z = jnp.zeros((L,), F32)
    for k in range(n // L):
        ref[pl.ds(k * L, L)] = z


def _zero_fill2d(ref, rows):
    z = jnp.zeros((L,), F32)
    for r in range(rows):
        ref[r, :] = z


def _mesh():
    return plsc.VectorSubcoreMesh(
        core_axis_name="c", subcore_axis_name="s", num_cores=NC,
        num_subcores=NS)


def _make_degrees(n_pad, nb0, nb1):
    sl = n_pad // NS

    @functools.partial(
        pl.kernel,
        out_type=jax.ShapeDtypeStruct((NC * 2 * n_pad,), F32),
        mesh=_mesh(),
        compiler_params=pltpu.CompilerParams(use_tc_tiling_on_sc=False),
        scratch_types=[
            pltpu.VMEM((CB, B), I32),
            pltpu.VMEM((CB, B), I32),
            pltpu.VMEM((B,), F32),
            pltpu.VMEM((sl,), F32),
            pltpu.VMEM_SHARED((n_pad,), F32),
            pltpu.VMEM_SHARED((n_pad,), F32),
            pltpu.SemaphoreType.DMA,
        ],
    )
    def deg_kernel(row2d, col2d, out, rbuf, cbuf, ones, zbuf, odeg, ideg,
                   sem):
        c = lax.axis_index("c")
        s = lax.axis_index("s")
        tb = jnp.where(c == 0, s * nb0, NS * nb0 + s * nb1)
        chunks = jnp.where(c == 0, nb0 // CB, nb1 // CB)
        _zero_fill(zbuf, sl)
        one = jnp.ones((L,), F32)
        for k in range(B // L):
            ones[pl.ds(k * L, L)] = one
        pltpu.sync_copy(zbuf, odeg.at[pl.ds(s * sl, sl)])
        pltpu.sync_copy(zbuf, ideg.at[pl.ds(s * sl, sl)])
        plsc.subcore_barrier()

        def chunk(ch, carry):
            base = tb + ch * CB
            pltpu.sync_copy(row2d.at[pl.ds(base, CB)], rbuf)
            pltpu.sync_copy(col2d.at[pl.ds(base, CB)], cbuf)
            sds = []
            for j in range(CB):
                sds.append(pltpu.async_copy(
                    ones, odeg.at[rbuf.at[j]], sem, add=True))
                sds.append(pltpu.async_copy(
                    ones, ideg.at[cbuf.at[j]], sem, add=True))
            for d in sds:
                d.wait()
            return carry

        lax.fori_loop(0, chunks, chunk, 0)
        plsc.subcore_barrier()
        pltpu.sync_copy(odeg.at[pl.ds(s * sl, sl)], zbuf)
        pltpu.sync_copy(zbuf, out.at[pl.ds(c * 2 * n_pad + s * sl, sl)])
        pltpu.sync_copy(ideg.at[pl.ds(s * sl, sl)], zbuf)
        pltpu.sync_copy(
            zbuf, out.at[pl.ds(c * 2 * n_pad + n_pad + s * sl, sl)])

    return deg_kernel


def _make_pass1(n_pad, c_dim, nb0, nb1):
    sl = n_pad // NS
    assert sl % ZR == 0

    @functools.partial(
        pl.kernel,
        out_type=jax.ShapeDtypeStruct((NC, n_pad // ZR, ZR, c_dim), F32),
        mesh=_mesh(),
        compiler_params=pltpu.CompilerParams(use_tc_tiling_on_sc=False),
        scratch_types=[
            pltpu.VMEM((CB, B), I32),
            pltpu.VMEM((CB, B), I32),
            pltpu.VMEM((CB, B, c_dim), F32),
            pltpu.VMEM((ZR, c_dim), F32),
            pltpu.VMEM_SHARED((n_pad, c_dim), F32),
            pltpu.SemaphoreType.DMA,
            pltpu.SemaphoreType.DMA,
        ],
    )
    def pass1_kernel(row2d, col2d, label_a, out, rbuf, cbuf, gbuf, zbuf,
                     acc, gsem, ssem):
        c = lax.axis_index("c")
        s = lax.axis_index("s")
        tb = jnp.where(c == 0, s * nb0, NS * nb0 + s * nb1)
        chunks = jnp.where(c == 0, nb0 // CB, nb1 // CB)
        _zero_fill2d(zbuf, ZR)
        for k in range(sl // ZR):
            pltpu.sync_copy(zbuf, acc.at[pl.ds(s * sl + k * ZR, ZR), :])
        plsc.subcore_barrier()

        def chunk(ch, carry):
            base = tb + ch * CB
            pltpu.sync_copy(row2d.at[pl.ds(base, CB)], rbuf)
            pltpu.sync_copy(col2d.at[pl.ds(base, CB)], cbuf)
            gds = [pltpu.async_copy(label_a.at[rbuf.at[j]], gbuf.at[j], gsem)
                   for j in range(CB)]
            for d in gds:
                d.wait()
            sds = [pltpu.async_copy(gbuf.at[j], acc.at[cbuf.at[j]], ssem,
                                    add=True)
                   for j in range(CB)]
            for d in sds:
                d.wait()
            return carry

        lax.fori_loop(0, chunks, chunk, 0)
        plsc.subcore_barrier()
        for k in range(sl // ZR):
            r0 = s * sl + k * ZR
            pltpu.sync_copy(acc.at[pl.ds(r0, ZR), :], zbuf)
            pltpu.sync_copy(zbuf, out.at[c, r0 // ZR])

    return pass1_kernel


def _make_pass2(n_pad, c_dim, nb0, nb1):
    sl = n_pad // NS
    assert sl % ZR == 0

    @functools.partial(
        pl.kernel,
        out_type=(jax.ShapeDtypeStruct((NC, n_pad // ZR, ZR, c_dim), F32),
                  jax.ShapeDtypeStruct((NC * n_pad,), F32)),
        mesh=_mesh(),
        compiler_params=pltpu.CompilerParams(use_tc_tiling_on_sc=False),
        scratch_types=[
            pltpu.VMEM((CB, B), I32),
            pltpu.VMEM((CB, B), I32),
            pltpu.VMEM((CB, B, c_dim), F32),
            pltpu.VMEM((CB, B), F32),
            pltpu.VMEM((ZR, c_dim), F32),
            pltpu.VMEM((sl,), F32),
            pltpu.VMEM_SHARED((n_pad, c_dim), F32),
            pltpu.VMEM_SHARED((n_pad,), F32),
            pltpu.SemaphoreType.DMA,
            pltpu.SemaphoreType.DMA,
        ],
    )
    def pass2_kernel(row2d, col2d, x1b, b2, t3out, treout, rbuf, cbuf, gbuf,
                     sbuf, zbuf, zs, acc, tre, gsem, ssem):
        c = lax.axis_index("c")
        s = lax.axis_index("s")
        tb = jnp.where(c == 0, s * nb0, NS * nb0 + s * nb1)
        chunks = jnp.where(c == 0, nb0 // CB, nb1 // CB)
        _zero_fill2d(zbuf, ZR)
        for k in range(sl // ZR):
            pltpu.sync_copy(zbuf, acc.at[pl.ds(s * sl + k * ZR, ZR), :])
        _zero_fill(zs, sl)
        pltpu.sync_copy(zs, tre.at[pl.ds(s * sl, sl)])
        plsc.subcore_barrier()

        def chunk(ch, carry):
            base = tb + ch * CB
            pltpu.sync_copy(row2d.at[pl.ds(base, CB)], rbuf)
            pltpu.sync_copy(col2d.at[pl.ds(base, CB)], cbuf)
            gds = []
            for j in range(CB):
                gds.append(pltpu.async_copy(
                    x1b.at[cbuf.at[j]], gbuf.at[j], gsem))
                gds.append(pltpu.async_copy(
                    b2.at[cbuf.at[j]], sbuf.at[j], gsem))
            for d in gds:
                d.wait()
            sds = []
            for j in range(CB):
                sds.append(pltpu.async_copy(
                    gbuf.at[j], acc.at[rbuf.at[j]], ssem, add=True))
                sds.append(pltpu.async_copy(
                    sbuf.at[j], tre.at[rbuf.at[j]], ssem, add=True))
            for d in sds:
                d.wait()
            return carry

        lax.fori_loop(0, chunks, chunk, 0)
        plsc.subcore_barrier()
        for k in range(sl // ZR):
            r0 = s * sl + k * ZR
            pltpu.sync_copy(acc.at[pl.ds(r0, ZR), :], zbuf)
            pltpu.sync_copy(zbuf, t3out.at[c, r0 // ZR])
        pltpu.sync_copy(tre.at[pl.ds(s * sl, sl)], zs)
        pltpu.sync_copy(zs, treout.at[pl.ds(c * n_pad + s * sl, sl)])

    return pass2_kernel


def _tc_scales(degpart, label_p, n_pad, c_dim, blk):
    def body(dp_ref, lab_ref, la_ref, a_ref, b_ref, a2_ref, b2_ref):
        dp = dp_ref[...]
        od = dp[0, 0, :] + dp[1, 0, :] + 1.0
        idg = dp[0, 1, :] + dp[1, 1, :] + 1.0
        a = lax.rsqrt(od)
        b = lax.rsqrt(idg)
        a_ref[...] = a[:, None]
        b_ref[...] = b[:, None]
        a2_ref[...] = (1.0 / od)[:, None]
        b2_ref[...] = (1.0 / idg)[:, None]
        la_ref[...] = a[:, None] * lab_ref[...]

    n_blk = n_pad // blk
    col = jax.ShapeDtypeStruct((n_pad, 1), F32)
    return pl.pallas_call(
        body,
        grid=(n_blk,),
        in_specs=[
            pl.BlockSpec((NC, 2, blk), lambda i: (0, 0, i)),
            pl.BlockSpec((blk, c_dim), lambda i: (i, 0)),
        ],
        out_specs=[
            pl.BlockSpec((blk, c_dim), lambda i: (i, 0)),
            pl.BlockSpec((blk, 1), lambda i: (i, 0)),
            pl.BlockSpec((blk, 1), lambda i: (i, 0)),
            pl.BlockSpec((blk, 1), lambda i: (i, 0)),
            pl.BlockSpec((blk, 1), lambda i: (i, 0)),
        ],
        out_shape=[jax.ShapeDtypeStruct((n_pad, c_dim), F32), col, col, col,
                   col],
    )(degpart, label_p)


def _tc_x1(t1part, b_col, b2_col, n_pad, c_dim, blk):
    def body(tp_ref, b_ref, b2_ref, x1_ref, x1b_ref):
        t1 = tp_ref[0] + tp_ref[1]
        x1_ref[...] = b_ref[...] * t1
        x1b_ref[...] = b2_ref[...] * t1

    n_blk = n_pad // blk
    return pl.pallas_call(
        body,
        grid=(n_blk,),
        in_specs=[
            pl.BlockSpec((NC, blk, c_dim), lambda i: (0, i, 0)),
            pl.BlockSpec((blk, 1), lambda i: (i, 0)),
            pl.BlockSpec((blk, 1), lambda i: (i, 0)),
        ],
        out_specs=[
            pl.BlockSpec((blk, c_dim), lambda i: (i, 0)),
            pl.BlockSpec((blk, c_dim), lambda i: (i, 0)),
        ],
        out_shape=[jax.ShapeDtypeStruct((n_pad, c_dim), F32),
                   jax.ShapeDtypeStruct((n_pad, c_dim), F32)],
    )(t1part, b_col, b2_col)


def _tc_finish(t3part, trepart, a_col, a2_col, x1, label_p, n_pad, c_dim,
               blk):
    def body(tp_ref, trp_ref, a_ref, a2_ref, x1_ref, lab_ref, o_ref):
        t3 = tp_ref[0] + tp_ref[1]
        tre = trp_ref[0] + trp_ref[1]
        x3 = a_ref[...] * t3 - lab_ref[...] * (a2_ref[...] * tre)
        x1 = x1_ref[...]
        o_ref[...] = jnp.concatenate([x3 - x1, x3, x1], axis=1)

    n_blk = n_pad // blk
    return pl.pallas_call(
        body,
        grid=(n_blk,),
        in_specs=[
            pl.BlockSpec((NC, blk, c_dim), lambda i: (0, i, 0)),
            pl.BlockSpec((NC, blk, 1), lambda i: (0, i, 0)),
            pl.BlockSpec((blk, 1), lambda i: (i, 0)),
            pl.BlockSpec((blk, 1), lambda i: (i, 0)),
            pl.BlockSpec((blk, c_dim), lambda i: (i, 0)),
            pl.BlockSpec((blk, c_dim), lambda i: (i, 0)),
        ],
        out_specs=pl.BlockSpec((blk, 3 * c_dim), lambda i: (i, 0)),
        out_shape=jax.ShapeDtypeStruct((n_pad, 3 * c_dim), F32),
    )(t3part, trepart, a_col, a2_col, x1, label_p)


def kernel(x, edge_index, edge_weight, label, is_direct):
    n = label.shape[0]
    c_dim = label.shape[1]
    e = edge_index.shape[1]

    nb_t = -(-e // (NW * B))
    nb_t = -(-nb_t // CB) * CB
    e_pad = NW * nb_t * B
    nb0 = int(0.70 * 2 * nb_t) // CB * CB
    nb1 = 2 * nb_t - nb0

    n_pad = -(-(n + 1) // (NS * ZR)) * (NS * ZR)

    row = edge_index[0]
    col = edge_index[1]
    pad = jnp.full((e_pad - e,), n, I32)
    row2d = jnp.concatenate([row, pad]).reshape(e_pad // B, B)
    col2d = jnp.concatenate([col, pad]).reshape(e_pad // B, B)
    label_p = jnp.zeros((n_pad, c_dim), F32).at[:n].set(label)

    blk = n_pad // 32

    degflat = _make_degrees(n_pad, nb0, nb1)(row2d, col2d)
    degpart = degflat.reshape(NC, 2, n_pad)
    label_a, a_col, b_col, a2_col, b2_col = _tc_scales(
        degpart, label_p, n_pad, c_dim, n_pad // 8)
    t1part = _make_pass1(n_pad, c_dim, nb0, nb1)(row2d, col2d, label_a)
    x1, x1b = _tc_x1(t1part.reshape(NC, n_pad, c_dim), b_col, b2_col,
                     n_pad, c_dim, blk)
    b2_flat = b2_col.reshape(n_pad)
    t3part, treflat = _make_pass2(n_pad, c_dim, nb0, nb1)(
        row2d, col2d, x1b, b2_flat)
    out = _tc_finish(t3part.reshape(NC, n_pad, c_dim),
                     treflat.reshape(NC, n_pad, 1), a_col, a2_col,
                     x1, label_p, n_pad, c_dim, blk)
    return out[:n]
